# baseline (speedup 1.0000x reference)
import math

import jax
import jax.numpy as jnp
from jax.experimental import pallas as pl
from jax.experimental.pallas import tpu as pltpu

_B = 4
_S = 4096
_D = 1024
_MAX_LEN = 4096
_TS = 512
_NS = _S // _TS
_LOG_FACTOR = -math.log(10000.0) / _D


def _dim_rows():
    d = jax.lax.broadcasted_iota(jnp.int32, (1, _D), 1)
    freq = jnp.exp(((d // 2) * 2).astype(jnp.float32) * _LOG_FACTOR)
    phase = (d % 2).astype(jnp.float32) * (math.pi / 2)
    return freq, phase


def _pe_kernel(x_ref, emb_ref, o_ref, v_ref, w_ref):
    s = pl.program_id(0)

    @pl.when(s == 0)
    def _init_pe():
        pos = jax.lax.broadcasted_iota(jnp.int32, (8, 1), 0).astype(jnp.float32)
        freq, phase = _dim_rows()
        angle = pos * freq + phase
        v_ref[0:8, :] = jnp.sin(angle)
        w_ref[0:8, :] = jnp.cos(angle)
        k = 8
        while k < _TS:
            c = jnp.cos(k * freq)
            sn = jnp.sin(k * freq)
            v0 = v_ref[0:k, :]
            w0 = w_ref[0:k, :]
            v_ref[k:2 * k, :] = v0 * c + w0 * sn
            w_ref[k:2 * k, :] = w0 * c - v0 * sn
            k *= 2

    @pl.when(s > 0)
    def _advance_pe():
        freq, _ = _dim_rows()
        c = jnp.cos(_TS * freq)
        sn = jnp.sin(_TS * freq)
        v = v_ref[...]
        w = w_ref[...]
        v_ref[...] = v * c + w * sn
        w_ref[...] = w * c - v * sn

    pe = v_ref[...] + emb_ref[0, :][None, :]
    o_ref[...] = x_ref[...] + pe[None, :, :]


def kernel(x, node_emb):
    return pl.pallas_call(
        _pe_kernel,
        grid=(_NS,),
        in_specs=[
            pl.BlockSpec((_B, _TS, _D), lambda s: (0, s, 0)),
            pl.BlockSpec((5, _D), lambda s: (0, 0)),
        ],
        out_specs=pl.BlockSpec((_B, _TS, _D), lambda s: (0, s, 0)),
        out_shape=jax.ShapeDtypeStruct((_B, _S, _D), jnp.float32),
        scratch_shapes=[
            pltpu.VMEM((_TS, _D), jnp.float32),
            pltpu.VMEM((_TS, _D), jnp.float32),
        ],
    )(x, node_emb)
